# baseline (speedup 1.0000x reference)
import functools

import jax
import jax.numpy as jnp
from jax import lax
from jax.experimental import pallas as pl
from jax.experimental.pallas import tpu as pltpu
from jax.experimental.pallas import tpu_sc as plsc

N = 10000
CIN = 128
NPAD = 10240
E = 320000
NC = 2
NS = 16
NW = NC * NS
CHUNK = 128
CH = 80
EPT = CH * CHUNK
EPAD = NW * EPT
ROWS_PER_TILE = NPAD // NS

_mesh = plsc.VectorSubcoreMesh(
    core_axis_name="c", subcore_axis_name="s", num_cores=NC, num_subcores=NS)


def _fill_rows(ref, nrows, ncolblk, value):
    v = jnp.full((16,), value, dtype=jnp.float32)

    def row(i, carry):
        for cb in range(ncolblk):
            ref[i, pl.ds(cb * 16, 16)] = v
        return carry

    lax.fori_loop(0, nrows, row, 0)


@functools.partial(
    pl.kernel,
    mesh=_mesh,
    out_type=jax.ShapeDtypeStruct((NC, NPAD, CIN), jnp.float32),
    scratch_types=[
        pltpu.VMEM((CH, CHUNK), jnp.int32),
        pltpu.VMEM((CHUNK, CIN), jnp.float32),
        pltpu.VMEM((CHUNK, CIN), jnp.float32),
        pltpu.VMEM_SHARED((NPAD, CIN), jnp.float32),
    ],
)
def _deg_kernel(dst_hbm, deg_hbm, dstv, ones_v, zeros_v, acc_sh):
    c = lax.axis_index("c")
    s = lax.axis_index("s")
    w = c * NS + s

    _fill_rows(ones_v, CHUNK, CIN // 16, 1.0)
    _fill_rows(zeros_v, CHUNK, CIN // 16, 0.0)
    for k in range(ROWS_PER_TILE // CHUNK):
        pltpu.sync_copy(zeros_v,
                        acc_sh.at[pl.ds(s * ROWS_PER_TILE + k * CHUNK, CHUNK)])
    plsc.subcore_barrier()

    pltpu.sync_copy(dst_hbm.at[w], dstv)

    def body(j, carry):
        pltpu.sync_copy(ones_v, acc_sh.at[dstv.at[j]], add=True)
        return carry

    lax.fori_loop(0, CH, body, 0)
    plsc.subcore_barrier()

    pltpu.sync_copy(acc_sh.at[pl.ds(s * ROWS_PER_TILE, ROWS_PER_TILE)],
                    deg_hbm.at[c, pl.ds(s * ROWS_PER_TILE, ROWS_PER_TILE)])


GRP = 8
NGRP = CH // GRP


@functools.partial(
    pl.kernel,
    mesh=_mesh,
    out_type=jax.ShapeDtypeStruct((NC, NPAD, CIN), jnp.float32),
    scratch_types=[
        pltpu.VMEM((GRP, CHUNK), jnp.int32),
        pltpu.VMEM((GRP, CHUNK), jnp.int32),
        pltpu.VMEM((GRP, CHUNK), jnp.int32),
        pltpu.VMEM((GRP, CHUNK), jnp.int32),
        pltpu.VMEM((CHUNK, CIN), jnp.float32),
        pltpu.VMEM((CHUNK, CIN), jnp.float32),
        pltpu.VMEM_SHARED((NPAD, CIN), jnp.float32),
        pltpu.SemaphoreType.DMA,
        pltpu.SemaphoreType.DMA,
        pltpu.SemaphoreType.DMA,
        pltpu.SemaphoreType.DMA,
        pltpu.SemaphoreType.DMA,
        pltpu.SemaphoreType.DMA,
    ],
)
def _spmm_kernel(y_hbm, src_hbm, dst_hbm, out_hbm, srcA, dstA, srcB, dstB,
                 rows0, rows1, acc_sh, sem0, sem1, semS0, semS1, semA, semB):
    c = lax.axis_index("c")
    s = lax.axis_index("s")
    w = c * NS + s

    _fill_rows(rows0, CHUNK, CIN // 16, 0.0)
    for k in range(ROWS_PER_TILE // CHUNK):
        pltpu.sync_copy(rows0,
                        acc_sh.at[pl.ds(s * ROWS_PER_TILE + k * CHUNK, CHUNK)])
    plsc.subcore_barrier()

    pltpu.sync_copy(src_hbm.at[w, pl.ds(0, GRP)], srcA)
    pltpu.sync_copy(dst_hbm.at[w, pl.ds(0, GRP)], dstA)
    pltpu.async_copy(src_hbm.at[w, pl.ds(GRP, GRP)], srcB, semB)
    pltpu.async_copy(dst_hbm.at[w, pl.ds(GRP, GRP)], dstB, semB)
    pltpu.async_copy(y_hbm.at[srcA.at[0]], rows0, sem0)
    pltpu.async_copy(y_hbm.at[srcA.at[1]], rows1, sem1)

    def _wait_idx(sC, dC, g, sem):
        pltpu.make_async_copy(src_hbm.at[w, pl.ds(g * GRP, GRP)], sC, sem).wait()
        pltpu.make_async_copy(dst_hbm.at[w, pl.ds(g * GRP, GRP)], dC, sem).wait()

    def _group(i, sC, dC, sN, dN, semN, last):
        for k in range(0, GRP, 2):
            pltpu.make_async_copy(y_hbm.at[sC.at[k]], rows0, sem0).wait()
            pltpu.async_copy(rows0, acc_sh.at[dC.at[k]], semS0, add=True)
            pltpu.make_async_copy(y_hbm.at[sC.at[k + 1]], rows1, sem1).wait()
            pltpu.async_copy(rows1, acc_sh.at[dC.at[k + 1]], semS1, add=True)

            pltpu.make_async_copy(rows0, acc_sh.at[dC.at[k]], semS0).wait()
            if k + 2 < GRP:
                pltpu.async_copy(y_hbm.at[sC.at[k + 2]], rows0, sem0)
            elif last is None:
                _wait_idx(sN, dN, 0, semN)
                pltpu.async_copy(y_hbm.at[sN.at[0]], rows0, sem0)
            else:

                @pl.when(i < last)
                def _():
                    _wait_idx(sN, dN, 0, semN)
                    pltpu.async_copy(y_hbm.at[sN.at[0]], rows0, sem0)

            pltpu.make_async_copy(rows1, acc_sh.at[dC.at[k + 1]], semS1).wait()
            if k + 3 < GRP:
                pltpu.async_copy(y_hbm.at[sC.at[k + 3]], rows1, sem1)
            elif last is None:
                pltpu.async_copy(y_hbm.at[sN.at[1]], rows1, sem1)
            else:

                @pl.when(i < last)
                def _():
                    pltpu.async_copy(y_hbm.at[sN.at[1]], rows1, sem1)

    def body(i, carry):
        _group(i, srcA, dstA, srcB, dstB, semB, None)

        @pl.when(i < NGRP // 2 - 1)
        def _():
            g = (i + 1) * 2
            pltpu.async_copy(src_hbm.at[w, pl.ds(g * GRP, GRP)], srcA, semA)
            pltpu.async_copy(dst_hbm.at[w, pl.ds(g * GRP, GRP)], dstA, semA)

        _group(i, srcB, dstB, srcA, dstA, semA, NGRP // 2 - 1)

        @pl.when(i < NGRP // 2 - 1)
        def _():
            g = (i + 1) * 2 + 1
            pltpu.async_copy(src_hbm.at[w, pl.ds(g * GRP, GRP)], srcB, semB)
            pltpu.async_copy(dst_hbm.at[w, pl.ds(g * GRP, GRP)], dstB, semB)

        return carry

    lax.fori_loop(0, NGRP // 2, body, 0)
    plsc.subcore_barrier()

    pltpu.sync_copy(acc_sh.at[pl.ds(s * ROWS_PER_TILE, ROWS_PER_TILE)],
                    out_hbm.at[c, pl.ds(s * ROWS_PER_TILE, ROWS_PER_TILE)])


def _prescale_body(degp_ref, x_ref, dis_ref, y_ref):
    deg = degp_ref[0, :, 0:1] + degp_ref[1, :, 0:1] + 1.0
    dis = lax.rsqrt(deg)
    dis_ref[...] = dis
    y_ref[...] = x_ref[...] * dis


def _layer1_body(parts_ref, y_ref, dis_ref, w_ref, b_ref, y2_ref):
    dis = dis_ref[...]
    z = (parts_ref[0] + parts_ref[1] + y_ref[...]) * dis
    h = jnp.dot(z, w_ref[...], preferred_element_type=jnp.float32) + b_ref[...]
    y2_ref[...] = jnp.maximum(h, 0.0) * dis


def _layer2_body(parts_ref, y_ref, dis_ref, w_ref, b_ref, wl_ref, bl_ref,
                 out_ref):
    dis = dis_ref[...]
    z = (parts_ref[0] + parts_ref[1] + y_ref[...]) * dis
    h = jnp.dot(z, w_ref[...], preferred_element_type=jnp.float32) + b_ref[...]
    h = jnp.maximum(h, 0.0)
    out_ref[...] = (jnp.dot(h, wl_ref[...], preferred_element_type=jnp.float32)
                    + bl_ref[...])


_BM = 512
_GRID = NPAD // _BM


def _tc_prescale(degp, xpad):
    return pl.pallas_call(
        _prescale_body,
        grid=(_GRID,),
        in_specs=[
            pl.BlockSpec((NC, _BM, CIN), lambda i: (0, i, 0)),
            pl.BlockSpec((_BM, CIN), lambda i: (i, 0)),
        ],
        out_specs=[
            pl.BlockSpec((_BM, 1), lambda i: (i, 0)),
            pl.BlockSpec((_BM, CIN), lambda i: (i, 0)),
        ],
        out_shape=[
            jax.ShapeDtypeStruct((NPAD, 1), jnp.float32),
            jax.ShapeDtypeStruct((NPAD, CIN), jnp.float32),
        ],
    )(degp, xpad)


def _tc_layer1(parts, y, dis, W, b):
    return pl.pallas_call(
        _layer1_body,
        grid=(_GRID,),
        in_specs=[
            pl.BlockSpec((NC, _BM, CIN), lambda i: (0, i, 0)),
            pl.BlockSpec((_BM, CIN), lambda i: (i, 0)),
            pl.BlockSpec((_BM, 1), lambda i: (i, 0)),
            pl.BlockSpec((CIN, CIN), lambda i: (0, 0)),
            pl.BlockSpec((1, CIN), lambda i: (0, 0)),
        ],
        out_specs=pl.BlockSpec((_BM, CIN), lambda i: (i, 0)),
        out_shape=jax.ShapeDtypeStruct((NPAD, CIN), jnp.float32),
    )(parts, y, dis, W, b)


def _tc_layer2(parts, y, dis, W, b, Wl, bl):
    return pl.pallas_call(
        _layer2_body,
        grid=(_GRID,),
        in_specs=[
            pl.BlockSpec((NC, _BM, CIN), lambda i: (0, i, 0)),
            pl.BlockSpec((_BM, CIN), lambda i: (i, 0)),
            pl.BlockSpec((_BM, 1), lambda i: (i, 0)),
            pl.BlockSpec((CIN, CIN), lambda i: (0, 0)),
            pl.BlockSpec((1, CIN), lambda i: (0, 0)),
            pl.BlockSpec((CIN, CIN), lambda i: (0, 0)),
            pl.BlockSpec((1, CIN), lambda i: (0, 0)),
        ],
        out_specs=pl.BlockSpec((_BM, CIN), lambda i: (i, 0)),
        out_shape=jax.ShapeDtypeStruct((NPAD, CIN), jnp.float32),
    )(parts, y, dis, W, b, Wl, bl)


def kernel(x, edge_index, W1, b1, W2, b2, Wl, bl):
    ei = edge_index.astype(jnp.int32)
    npad = EPAD - E
    padidx = N + (jnp.arange(npad, dtype=jnp.int32) % (NPAD - N))
    src3 = jnp.concatenate([ei[0], padidx]).reshape(NW, CH, CHUNK)
    dst3 = jnp.concatenate([ei[1], padidx]).reshape(NW, CH, CHUNK)
    xpad = jnp.pad(x, ((0, NPAD - N), (0, 0)))

    degp = _deg_kernel(dst3)
    dis, y1 = _tc_prescale(degp, xpad)
    p1 = _spmm_kernel(y1, src3, dst3)
    y2 = _tc_layer1(p1, y1, dis, W1, b1.reshape(1, CIN))
    p2 = _spmm_kernel(y2, src3, dst3)
    out = _tc_layer2(p2, y2, dis, W2, b2.reshape(1, CIN), Wl,
                     bl.reshape(1, CIN))
    return out[:N]


# speedup vs baseline: 1.2759x; 1.2759x over previous
import functools

import jax
import jax.numpy as jnp
from jax import lax
from jax.experimental import pallas as pl
from jax.experimental.pallas import tpu as pltpu
from jax.experimental.pallas import tpu_sc as plsc

N = 10000
CIN = 128
NPAD = 10240
E = 320000
NC = 2
NS = 16
NW = NC * NS
CHUNK = 128
CH = 80
EPT = CH * CHUNK
EPAD = NW * EPT
ROWS_PER_TILE = NPAD // NS

_mesh = plsc.VectorSubcoreMesh(
    core_axis_name="c", subcore_axis_name="s", num_cores=NC, num_subcores=NS)


def _fill_rows(ref, nrows, ncolblk, value):
    v = jnp.full((16,), value, dtype=jnp.float32)

    def row(i, carry):
        for cb in range(ncolblk):
            ref[i, pl.ds(cb * 16, 16)] = v
        return carry

    lax.fori_loop(0, nrows, row, 0)


@functools.partial(
    pl.kernel,
    mesh=_mesh,
    out_type=jax.ShapeDtypeStruct((NC, NPAD, CIN), jnp.float32),
    scratch_types=[
        pltpu.VMEM((CH, CHUNK), jnp.int32),
        pltpu.VMEM((CHUNK, CIN), jnp.float32),
        pltpu.VMEM((CHUNK, CIN), jnp.float32),
        pltpu.VMEM_SHARED((NPAD, CIN), jnp.float32),
    ],
)
def _deg_kernel(dst_hbm, deg_hbm, dstv, ones_v, zeros_v, acc_sh):
    c = lax.axis_index("c")
    s = lax.axis_index("s")
    w = c * NS + s

    _fill_rows(ones_v, CHUNK, CIN // 16, 1.0)
    _fill_rows(zeros_v, CHUNK, CIN // 16, 0.0)
    for k in range(ROWS_PER_TILE // CHUNK):
        pltpu.sync_copy(zeros_v,
                        acc_sh.at[pl.ds(s * ROWS_PER_TILE + k * CHUNK, CHUNK)])
    plsc.subcore_barrier()

    pltpu.sync_copy(dst_hbm.at[w], dstv)

    def body(j, carry):
        pltpu.sync_copy(ones_v, acc_sh.at[dstv.at[j]], add=True)
        return carry

    lax.fori_loop(0, CH, body, 0)
    plsc.subcore_barrier()

    pltpu.sync_copy(acc_sh.at[pl.ds(s * ROWS_PER_TILE, ROWS_PER_TILE)],
                    deg_hbm.at[c, pl.ds(s * ROWS_PER_TILE, ROWS_PER_TILE)])


GRP = 8
NGRP = CH // GRP


@functools.partial(
    pl.kernel,
    mesh=_mesh,
    out_type=jax.ShapeDtypeStruct((NC, NPAD, CIN), jnp.float32),
    scratch_types=[
        pltpu.VMEM((GRP, CHUNK), jnp.int32),
        pltpu.VMEM((GRP, CHUNK), jnp.int32),
        pltpu.VMEM((GRP, CHUNK), jnp.int32),
        pltpu.VMEM((GRP, CHUNK), jnp.int32),
        pltpu.VMEM((CHUNK, CIN), jnp.float32),
        pltpu.VMEM((CHUNK, CIN), jnp.float32),
        pltpu.VMEM_SHARED((NPAD, CIN), jnp.float32),
        pltpu.SemaphoreType.DMA,
        pltpu.SemaphoreType.DMA,
        pltpu.SemaphoreType.DMA,
        pltpu.SemaphoreType.DMA,
    ],
)
def _spmm_kernel(y_hbm, src_hbm, dst_hbm, out_hbm, srcA, dstA, srcB, dstB,
                 rows0, rows1, acc_sh, sem0, sem1, semA, semB):
    c = lax.axis_index("c")
    s = lax.axis_index("s")
    w = c * NS + s

    _fill_rows(rows0, CHUNK, CIN // 16, 0.0)
    for k in range(ROWS_PER_TILE // CHUNK):
        pltpu.sync_copy(rows0,
                        acc_sh.at[pl.ds(s * ROWS_PER_TILE + k * CHUNK, CHUNK)])
    plsc.subcore_barrier()

    pltpu.sync_copy(src_hbm.at[w, pl.ds(0, GRP)], srcA)
    pltpu.sync_copy(dst_hbm.at[w, pl.ds(0, GRP)], dstA)
    pltpu.async_copy(src_hbm.at[w, pl.ds(GRP, GRP)], srcB, semB)
    pltpu.async_copy(dst_hbm.at[w, pl.ds(GRP, GRP)], dstB, semB)
    pltpu.async_copy(y_hbm.at[srcA.at[0]], rows0, sem0)

    def _wait_idx(sC, dC, g, sem):
        pltpu.make_async_copy(src_hbm.at[w, pl.ds(g * GRP, GRP)], sC, sem).wait()
        pltpu.make_async_copy(dst_hbm.at[w, pl.ds(g * GRP, GRP)], dC, sem).wait()

    def _group(i, sC, dC, sN, dN, semN, last):
        for k in range(0, GRP, 2):
            pltpu.async_copy(y_hbm.at[sC.at[k + 1]], rows1, sem1)
            pltpu.make_async_copy(y_hbm.at[sC.at[k]], rows0, sem0).wait()
            pltpu.sync_copy(rows0, acc_sh.at[dC.at[k]], add=True)
            if k + 2 < GRP:
                pltpu.async_copy(y_hbm.at[sC.at[k + 2]], rows0, sem0)
            elif last is None:
                _wait_idx(sN, dN, 0, semN)
                pltpu.async_copy(y_hbm.at[sN.at[0]], rows0, sem0)
            else:

                @pl.when(i < last)
                def _():
                    _wait_idx(sN, dN, 0, semN)
                    pltpu.async_copy(y_hbm.at[sN.at[0]], rows0, sem0)

            pltpu.make_async_copy(y_hbm.at[sC.at[k + 1]], rows1, sem1).wait()
            pltpu.sync_copy(rows1, acc_sh.at[dC.at[k + 1]], add=True)

    def body(i, carry):
        _group(i, srcA, dstA, srcB, dstB, semB, None)

        @pl.when(i < NGRP // 2 - 1)
        def _():
            g = (i + 1) * 2
            pltpu.async_copy(src_hbm.at[w, pl.ds(g * GRP, GRP)], srcA, semA)
            pltpu.async_copy(dst_hbm.at[w, pl.ds(g * GRP, GRP)], dstA, semA)

        _group(i, srcB, dstB, srcA, dstA, semA, NGRP // 2 - 1)

        @pl.when(i < NGRP // 2 - 1)
        def _():
            g = (i + 1) * 2 + 1
            pltpu.async_copy(src_hbm.at[w, pl.ds(g * GRP, GRP)], srcB, semB)
            pltpu.async_copy(dst_hbm.at[w, pl.ds(g * GRP, GRP)], dstB, semB)

        return carry

    lax.fori_loop(0, NGRP // 2, body, 0)
    plsc.subcore_barrier()

    pltpu.sync_copy(acc_sh.at[pl.ds(s * ROWS_PER_TILE, ROWS_PER_TILE)],
                    out_hbm.at[c, pl.ds(s * ROWS_PER_TILE, ROWS_PER_TILE)])


def _prescale_body(degp_ref, x_ref, dis_ref, y_ref):
    deg = degp_ref[0, :, 0:1] + degp_ref[1, :, 0:1] + 1.0
    dis = lax.rsqrt(deg)
    dis_ref[...] = dis
    y_ref[...] = x_ref[...] * dis


def _layer1_body(parts_ref, y_ref, dis_ref, w_ref, b_ref, y2_ref):
    dis = dis_ref[...]
    z = (parts_ref[0] + parts_ref[1] + y_ref[...]) * dis
    h = jnp.dot(z, w_ref[...], preferred_element_type=jnp.float32) + b_ref[...]
    y2_ref[...] = jnp.maximum(h, 0.0) * dis


def _layer2_body(parts_ref, y_ref, dis_ref, w_ref, b_ref, wl_ref, bl_ref,
                 out_ref):
    dis = dis_ref[...]
    z = (parts_ref[0] + parts_ref[1] + y_ref[...]) * dis
    h = jnp.dot(z, w_ref[...], preferred_element_type=jnp.float32) + b_ref[...]
    h = jnp.maximum(h, 0.0)
    out_ref[...] = (jnp.dot(h, wl_ref[...], preferred_element_type=jnp.float32)
                    + bl_ref[...])


_BM = 1024
_GRID = NPAD // _BM


def _tc_prescale(degp, xpad):
    return pl.pallas_call(
        _prescale_body,
        grid=(_GRID,),
        in_specs=[
            pl.BlockSpec((NC, _BM, CIN), lambda i: (0, i, 0)),
            pl.BlockSpec((_BM, CIN), lambda i: (i, 0)),
        ],
        out_specs=[
            pl.BlockSpec((_BM, 1), lambda i: (i, 0)),
            pl.BlockSpec((_BM, CIN), lambda i: (i, 0)),
        ],
        out_shape=[
            jax.ShapeDtypeStruct((NPAD, 1), jnp.float32),
            jax.ShapeDtypeStruct((NPAD, CIN), jnp.float32),
        ],
    )(degp, xpad)


def _tc_layer1(parts, y, dis, W, b):
    return pl.pallas_call(
        _layer1_body,
        grid=(_GRID,),
        in_specs=[
            pl.BlockSpec((NC, _BM, CIN), lambda i: (0, i, 0)),
            pl.BlockSpec((_BM, CIN), lambda i: (i, 0)),
            pl.BlockSpec((_BM, 1), lambda i: (i, 0)),
            pl.BlockSpec((CIN, CIN), lambda i: (0, 0)),
            pl.BlockSpec((1, CIN), lambda i: (0, 0)),
        ],
        out_specs=pl.BlockSpec((_BM, CIN), lambda i: (i, 0)),
        out_shape=jax.ShapeDtypeStruct((NPAD, CIN), jnp.float32),
    )(parts, y, dis, W, b)


def _tc_layer2(parts, y, dis, W, b, Wl, bl):
    bm2 = 1000
    return pl.pallas_call(
        _layer2_body,
        grid=(N // bm2,),
        in_specs=[
            pl.BlockSpec((NC, bm2, CIN), lambda i: (0, i, 0)),
            pl.BlockSpec((bm2, CIN), lambda i: (i, 0)),
            pl.BlockSpec((bm2, 1), lambda i: (i, 0)),
            pl.BlockSpec((CIN, CIN), lambda i: (0, 0)),
            pl.BlockSpec((1, CIN), lambda i: (0, 0)),
            pl.BlockSpec((CIN, CIN), lambda i: (0, 0)),
            pl.BlockSpec((1, CIN), lambda i: (0, 0)),
        ],
        out_specs=pl.BlockSpec((bm2, CIN), lambda i: (i, 0)),
        out_shape=jax.ShapeDtypeStruct((N, CIN), jnp.float32),
    )(parts, y, dis, W, b, Wl, bl)


def kernel(x, edge_index, W1, b1, W2, b2, Wl, bl):
    ei = edge_index.astype(jnp.int32)
    npad = EPAD - E
    padidx = N + (jnp.arange(npad, dtype=jnp.int32) % (NPAD - N))
    src3 = jnp.concatenate([ei[0], padidx]).reshape(NW, CH, CHUNK)
    dst3 = jnp.concatenate([ei[1], padidx]).reshape(NW, CH, CHUNK)
    xpad = jnp.pad(x, ((0, NPAD - N), (0, 0)))

    degp = _deg_kernel(dst3)
    dis, y1 = _tc_prescale(degp, xpad)
    p1 = _spmm_kernel(y1, src3, dst3)
    y2 = _tc_layer1(p1, y1, dis, W1, b1.reshape(1, CIN))
    p2 = _spmm_kernel(y2, src3, dst3)
    return _tc_layer2(p2, y2, dis, W2, b2.reshape(1, CIN), Wl,
                      bl.reshape(1, CIN))


# speedup vs baseline: 1.2820x; 1.0048x over previous
import functools

import jax
import jax.numpy as jnp
from jax import lax
from jax.experimental import pallas as pl
from jax.experimental.pallas import tpu as pltpu
from jax.experimental.pallas import tpu_sc as plsc

N = 10000
CIN = 128
NPAD = 10240
E = 320000
NC = 2
NS = 16
NW = NC * NS
CHUNK = 128
CH = 80
EPT = CH * CHUNK
EPAD = NW * EPT
ROWS_PER_TILE = NPAD // NS
OUT_ROWS_PER_TILE = N // NS

_mesh = plsc.VectorSubcoreMesh(
    core_axis_name="c", subcore_axis_name="s", num_cores=NC, num_subcores=NS)


def _fill_rows(ref, nrows, ncolblk, value, dtype=jnp.float32):
    if dtype == jnp.int16:
        v = jnp.full((2, 16), value, dtype=dtype)

        def row(i, carry):
            for cb in range(ncolblk * 2):
                ref[pl.ds(i * 2, 2), pl.ds(cb * 16, 16)] = v
            return carry

        lax.fori_loop(0, nrows // 2, row, 0)
    else:
        v = jnp.full((16,), value, dtype=dtype)

        def row(i, carry):
            for cb in range(ncolblk):
                ref[i, pl.ds(cb * 16, 16)] = v
            return carry

        lax.fori_loop(0, nrows, row, 0)


def _copy_out_rows(acc_sh, out_hbm, c, s):
    base = pl.multiple_of(s * 624, 8)

    @pl.when(s < NS - 1)
    def _():
        pltpu.sync_copy(acc_sh.at[pl.ds(base, 624)],
                        out_hbm.at[c, pl.ds(base, 624)])

    @pl.when(s == NS - 1)
    def _():
        pltpu.sync_copy(acc_sh.at[pl.ds(15 * 624, N - 15 * 624)],
                        out_hbm.at[c, pl.ds(15 * 624, N - 15 * 624)])


@functools.partial(
    pl.kernel,
    mesh=_mesh,
    out_type=jax.ShapeDtypeStruct((NC, N, CIN), jnp.float32),
    scratch_types=[
        pltpu.VMEM((CH, CHUNK), jnp.int32),
        pltpu.VMEM((CHUNK, CIN), jnp.float32),
        pltpu.VMEM((CHUNK, CIN), jnp.float32),
        pltpu.VMEM_SHARED((NPAD, CIN), jnp.float32),
    ],
)
def _deg_kernel(dst_hbm, deg_hbm, dstv, ones_v, zeros_v, acc_sh):
    c = lax.axis_index("c")
    s = lax.axis_index("s")
    w = c * NS + s

    _fill_rows(ones_v, CHUNK, CIN // 16, 1.0)
    _fill_rows(zeros_v, CHUNK, CIN // 16, 0.0)
    for k in range(ROWS_PER_TILE // CHUNK):
        pltpu.sync_copy(zeros_v,
                        acc_sh.at[pl.ds(s * ROWS_PER_TILE + k * CHUNK, CHUNK)])
    plsc.subcore_barrier()

    pltpu.sync_copy(dst_hbm.at[w], dstv)

    def body(j, carry):
        pltpu.sync_copy(ones_v, acc_sh.at[dstv.at[j]], add=True)
        return carry

    lax.fori_loop(0, CH, body, 0)
    plsc.subcore_barrier()

    _copy_out_rows(acc_sh, deg_hbm, c, s)


GRP = 8
NGRP = CH // GRP


@functools.partial(
    pl.kernel,
    mesh=_mesh,
    out_type=jax.ShapeDtypeStruct((NC, N, CIN), jnp.float32),
    scratch_types=[
        pltpu.VMEM((GRP, CHUNK), jnp.int32),
        pltpu.VMEM((GRP, CHUNK), jnp.int32),
        pltpu.VMEM((GRP, CHUNK), jnp.int32),
        pltpu.VMEM((GRP, CHUNK), jnp.int32),
        pltpu.VMEM((CHUNK, CIN), jnp.float32),
        pltpu.VMEM((CHUNK, CIN), jnp.float32),
        pltpu.VMEM_SHARED((NPAD, CIN), jnp.float32),
        pltpu.SemaphoreType.DMA,
        pltpu.SemaphoreType.DMA,
        pltpu.SemaphoreType.DMA,
        pltpu.SemaphoreType.DMA,
    ],
)
def _spmm_kernel(y_hbm, src_hbm, dst_hbm, out_hbm, srcA, dstA, srcB, dstB,
                 rows0, rows1, acc_sh, sem0, sem1, semA, semB):
    c = lax.axis_index("c")
    s = lax.axis_index("s")
    w = c * NS + s

    _fill_rows(rows0, CHUNK, CIN // 16, 0.0)
    for k in range(ROWS_PER_TILE // CHUNK):
        pltpu.sync_copy(rows0,
                        acc_sh.at[pl.ds(s * ROWS_PER_TILE + k * CHUNK, CHUNK)])
    plsc.subcore_barrier()

    pltpu.sync_copy(src_hbm.at[w, pl.ds(0, GRP)], srcA)
    pltpu.sync_copy(dst_hbm.at[w, pl.ds(0, GRP)], dstA)
    pltpu.async_copy(src_hbm.at[w, pl.ds(GRP, GRP)], srcB, semB)
    pltpu.async_copy(dst_hbm.at[w, pl.ds(GRP, GRP)], dstB, semB)
    pltpu.async_copy(y_hbm.at[srcA.at[0]], rows0, sem0)

    def _wait_idx(sC, dC, g, sem):
        pltpu.make_async_copy(src_hbm.at[w, pl.ds(g * GRP, GRP)], sC, sem).wait()
        pltpu.make_async_copy(dst_hbm.at[w, pl.ds(g * GRP, GRP)], dC, sem).wait()

    def _group(i, sC, dC, sN, dN, semN, last):
        for k in range(0, GRP, 2):
            pltpu.async_copy(y_hbm.at[sC.at[k + 1]], rows1, sem1)
            pltpu.make_async_copy(y_hbm.at[sC.at[k]], rows0, sem0).wait()
            pltpu.sync_copy(rows0, acc_sh.at[dC.at[k]], add=True)
            if k + 2 < GRP:
                pltpu.async_copy(y_hbm.at[sC.at[k + 2]], rows0, sem0)
            elif last is None:
                _wait_idx(sN, dN, 0, semN)
                pltpu.async_copy(y_hbm.at[sN.at[0]], rows0, sem0)
            else:

                @pl.when(i < last)
                def _():
                    _wait_idx(sN, dN, 0, semN)
                    pltpu.async_copy(y_hbm.at[sN.at[0]], rows0, sem0)

            pltpu.make_async_copy(y_hbm.at[sC.at[k + 1]], rows1, sem1).wait()
            pltpu.sync_copy(rows1, acc_sh.at[dC.at[k + 1]], add=True)

    def body(i, carry):
        _group(i, srcA, dstA, srcB, dstB, semB, None)

        @pl.when(i < NGRP // 2 - 1)
        def _():
            g = (i + 1) * 2
            pltpu.async_copy(src_hbm.at[w, pl.ds(g * GRP, GRP)], srcA, semA)
            pltpu.async_copy(dst_hbm.at[w, pl.ds(g * GRP, GRP)], dstA, semA)

        _group(i, srcB, dstB, srcA, dstA, semA, NGRP // 2 - 1)

        @pl.when(i < NGRP // 2 - 1)
        def _():
            g = (i + 1) * 2 + 1
            pltpu.async_copy(src_hbm.at[w, pl.ds(g * GRP, GRP)], srcB, semB)
            pltpu.async_copy(dst_hbm.at[w, pl.ds(g * GRP, GRP)], dstB, semB)

        return carry

    lax.fori_loop(0, NGRP // 2, body, 0)
    plsc.subcore_barrier()

    _copy_out_rows(acc_sh, out_hbm, c, s)


def _prescale_body(degp_ref, x_ref, dis_ref, y_ref):
    deg = degp_ref[0, :, 0:1] + degp_ref[1, :, 0:1] + 1.0
    dis = lax.rsqrt(deg)
    dis_ref[...] = dis
    y_ref[...] = x_ref[...] * dis


def _layer1_body(parts_ref, y_ref, dis_ref, w_ref, b_ref, y2_ref):
    dis = dis_ref[...]
    z = (parts_ref[0] + parts_ref[1] + y_ref[...]) * dis
    h = jnp.dot(z, w_ref[...], preferred_element_type=jnp.float32) + b_ref[...]
    y2_ref[...] = jnp.maximum(h, 0.0) * dis


def _layer2_body(parts_ref, y_ref, dis_ref, w_ref, b_ref, wl_ref, bl_ref,
                 out_ref):
    dis = dis_ref[...]
    z = (parts_ref[0] + parts_ref[1] + y_ref[...]) * dis
    h = jnp.dot(z, w_ref[...], preferred_element_type=jnp.float32) + b_ref[...]
    h = jnp.maximum(h, 0.0)
    out_ref[...] = (jnp.dot(h, wl_ref[...], preferred_element_type=jnp.float32)
                    + bl_ref[...])


_BM = 1000
_GRID = N // _BM


def _tc_prescale(degp, xpad):
    return pl.pallas_call(
        _prescale_body,
        grid=(_GRID,),
        in_specs=[
            pl.BlockSpec((NC, _BM, CIN), lambda i: (0, i, 0)),
            pl.BlockSpec((_BM, CIN), lambda i: (i, 0)),
        ],
        out_specs=[
            pl.BlockSpec((_BM, 1), lambda i: (i, 0)),
            pl.BlockSpec((_BM, CIN), lambda i: (i, 0)),
        ],
        out_shape=[
            jax.ShapeDtypeStruct((N, 1), jnp.float32),
            jax.ShapeDtypeStruct((N, CIN), jnp.float32),
        ],
    )(degp, xpad)


def _tc_layer1(parts, y, dis, W, b):
    return pl.pallas_call(
        _layer1_body,
        grid=(_GRID,),
        in_specs=[
            pl.BlockSpec((NC, _BM, CIN), lambda i: (0, i, 0)),
            pl.BlockSpec((_BM, CIN), lambda i: (i, 0)),
            pl.BlockSpec((_BM, 1), lambda i: (i, 0)),
            pl.BlockSpec((CIN, CIN), lambda i: (0, 0)),
            pl.BlockSpec((1, CIN), lambda i: (0, 0)),
        ],
        out_specs=pl.BlockSpec((_BM, CIN), lambda i: (i, 0)),
        out_shape=jax.ShapeDtypeStruct((N, CIN), jnp.float32),
    )(parts, y, dis, W, b)


def _tc_layer2(parts, y, dis, W, b, Wl, bl):
    return pl.pallas_call(
        _layer2_body,
        grid=(_GRID,),
        in_specs=[
            pl.BlockSpec((NC, _BM, CIN), lambda i: (0, i, 0)),
            pl.BlockSpec((_BM, CIN), lambda i: (i, 0)),
            pl.BlockSpec((_BM, 1), lambda i: (i, 0)),
            pl.BlockSpec((CIN, CIN), lambda i: (0, 0)),
            pl.BlockSpec((1, CIN), lambda i: (0, 0)),
            pl.BlockSpec((CIN, CIN), lambda i: (0, 0)),
            pl.BlockSpec((1, CIN), lambda i: (0, 0)),
        ],
        out_specs=pl.BlockSpec((_BM, CIN), lambda i: (i, 0)),
        out_shape=jax.ShapeDtypeStruct((N, CIN), jnp.float32),
    )(parts, y, dis, W, b, Wl, bl)


def kernel(x, edge_index, W1, b1, W2, b2, Wl, bl):
    ei = edge_index.astype(jnp.int32)
    npad = EPAD - E
    pad_src = jnp.arange(npad, dtype=jnp.int32) % N
    pad_dst = N + (jnp.arange(npad, dtype=jnp.int32) % (NPAD - N))
    src3 = jnp.concatenate([ei[0], pad_src]).reshape(NW, CH, CHUNK)
    dst3 = jnp.concatenate([ei[1], pad_dst]).reshape(NW, CH, CHUNK)

    degp = _deg_kernel(dst3)
    dis, y1 = _tc_prescale(degp, x)
    p1 = _spmm_kernel(y1, src3, dst3)
    y2 = _tc_layer1(p1, y1, dis, W1, b1.reshape(1, CIN))
    p2 = _spmm_kernel(y2, src3, dst3)
    return _tc_layer2(p2, y2, dis, W2, b2.reshape(1, CIN), Wl,
                      bl.reshape(1, CIN))


# speedup vs baseline: 1.2952x; 1.0103x over previous
import functools

import jax
import jax.numpy as jnp
from jax import lax
from jax.experimental import pallas as pl
from jax.experimental.pallas import tpu as pltpu
from jax.experimental.pallas import tpu_sc as plsc

N = 10000
CIN = 128
NPAD = 10240
E = 320000
NC = 2
NS = 16
NW = NC * NS
CHUNK = 128
CH = 80
EPT = CH * CHUNK
EPAD = NW * EPT
ROWS_PER_TILE = NPAD // NS
OUT_ROWS_PER_TILE = N // NS

_mesh = plsc.VectorSubcoreMesh(
    core_axis_name="c", subcore_axis_name="s", num_cores=NC, num_subcores=NS)


def _fill_rows(ref, nrows, ncolblk, value, dtype=jnp.float32):
    if dtype == jnp.int16:
        v = jnp.full((2, 16), value, dtype=dtype)

        def row(i, carry):
            for cb in range(ncolblk * 2):
                ref[pl.ds(i * 2, 2), pl.ds(cb * 16, 16)] = v
            return carry

        lax.fori_loop(0, nrows // 2, row, 0)
    else:
        v = jnp.full((16,), value, dtype=dtype)

        def row(i, carry):
            for cb in range(ncolblk):
                ref[i, pl.ds(cb * 16, 16)] = v
            return carry

        lax.fori_loop(0, nrows, row, 0)


def _copy_out_rows(acc_sh, out_hbm, c, s):
    base = pl.multiple_of(s * 624, 8)

    @pl.when(s < NS - 1)
    def _():
        pltpu.sync_copy(acc_sh.at[pl.ds(base, 624)],
                        out_hbm.at[c, pl.ds(base, 624)])

    @pl.when(s == NS - 1)
    def _():
        pltpu.sync_copy(acc_sh.at[pl.ds(15 * 624, N - 15 * 624)],
                        out_hbm.at[c, pl.ds(15 * 624, N - 15 * 624)])


@functools.partial(
    pl.kernel,
    mesh=_mesh,
    out_type=jax.ShapeDtypeStruct((NC, N, CIN), jnp.float32),
    scratch_types=[
        pltpu.VMEM((CH, CHUNK), jnp.int32),
        pltpu.VMEM((CHUNK, CIN), jnp.float32),
        pltpu.VMEM((CHUNK, CIN), jnp.float32),
        pltpu.VMEM_SHARED((NPAD, CIN), jnp.float32),
        pltpu.SemaphoreType.DMA,
    ],
)
def _deg_kernel(dst_hbm, deg_hbm, dstv, ones_v, zeros_v, acc_sh, semz):
    c = lax.axis_index("c")
    s = lax.axis_index("s")
    w = c * NS + s

    _fill_rows(ones_v, CHUNK, CIN // 16, 1.0)
    _fill_rows(zeros_v, CHUNK, CIN // 16, 0.0)
    for k in range(ROWS_PER_TILE // CHUNK):
        pltpu.async_copy(
            zeros_v, acc_sh.at[pl.ds(s * ROWS_PER_TILE + k * CHUNK, CHUNK)],
            semz)
    pltpu.sync_copy(dst_hbm.at[w], dstv)
    for k in range(ROWS_PER_TILE // CHUNK):
        pltpu.make_async_copy(
            zeros_v, acc_sh.at[pl.ds(s * ROWS_PER_TILE + k * CHUNK, CHUNK)],
            semz).wait()
    plsc.subcore_barrier()

    def body(j, carry):
        pltpu.sync_copy(ones_v, acc_sh.at[dstv.at[j]], add=True)
        return carry

    lax.fori_loop(0, CH, body, 0)
    plsc.subcore_barrier()

    _copy_out_rows(acc_sh, deg_hbm, c, s)


GRP = 8
NGRP = CH // GRP


@functools.partial(
    pl.kernel,
    mesh=_mesh,
    out_type=jax.ShapeDtypeStruct((NC, N, CIN), jnp.float32),
    scratch_types=[
        pltpu.VMEM((GRP, CHUNK), jnp.int32),
        pltpu.VMEM((GRP, CHUNK), jnp.int32),
        pltpu.VMEM((GRP, CHUNK), jnp.int32),
        pltpu.VMEM((GRP, CHUNK), jnp.int32),
        pltpu.VMEM((CHUNK, CIN), jnp.float32),
        pltpu.VMEM((CHUNK, CIN), jnp.float32),
        pltpu.VMEM_SHARED((NPAD, CIN), jnp.float32),
        pltpu.SemaphoreType.DMA,
        pltpu.SemaphoreType.DMA,
        pltpu.SemaphoreType.DMA,
        pltpu.SemaphoreType.DMA,
    ],
)
def _spmm_kernel(y_hbm, src_hbm, dst_hbm, out_hbm, srcA, dstA, srcB, dstB,
                 rows0, rows1, acc_sh, sem0, sem1, semA, semB):
    c = lax.axis_index("c")
    s = lax.axis_index("s")
    w = c * NS + s

    _fill_rows(rows0, CHUNK, CIN // 16, 0.0)
    for k in range(ROWS_PER_TILE // CHUNK):
        pltpu.async_copy(
            rows0, acc_sh.at[pl.ds(s * ROWS_PER_TILE + k * CHUNK, CHUNK)],
            semA)
    pltpu.sync_copy(src_hbm.at[w, pl.ds(0, GRP)], srcA)
    pltpu.sync_copy(dst_hbm.at[w, pl.ds(0, GRP)], dstA)
    pltpu.async_copy(src_hbm.at[w, pl.ds(GRP, GRP)], srcB, semB)
    pltpu.async_copy(dst_hbm.at[w, pl.ds(GRP, GRP)], dstB, semB)
    for k in range(ROWS_PER_TILE // CHUNK):
        pltpu.make_async_copy(
            rows0, acc_sh.at[pl.ds(s * ROWS_PER_TILE + k * CHUNK, CHUNK)],
            semA).wait()
    pltpu.async_copy(y_hbm.at[srcA.at[0]], rows0, sem0)
    plsc.subcore_barrier()

    def _wait_idx(sC, dC, g, sem):
        pltpu.make_async_copy(src_hbm.at[w, pl.ds(g * GRP, GRP)], sC, sem).wait()
        pltpu.make_async_copy(dst_hbm.at[w, pl.ds(g * GRP, GRP)], dC, sem).wait()

    def _group(i, sC, dC, sN, dN, semN, last):
        for k in range(0, GRP, 2):
            pltpu.async_copy(y_hbm.at[sC.at[k + 1]], rows1, sem1)
            pltpu.make_async_copy(y_hbm.at[sC.at[k]], rows0, sem0).wait()
            pltpu.sync_copy(rows0, acc_sh.at[dC.at[k]], add=True)
            if k + 2 < GRP:
                pltpu.async_copy(y_hbm.at[sC.at[k + 2]], rows0, sem0)
            elif last is None:
                _wait_idx(sN, dN, 0, semN)
                pltpu.async_copy(y_hbm.at[sN.at[0]], rows0, sem0)
            else:

                @pl.when(i < last)
                def _():
                    _wait_idx(sN, dN, 0, semN)
                    pltpu.async_copy(y_hbm.at[sN.at[0]], rows0, sem0)

            pltpu.make_async_copy(y_hbm.at[sC.at[k + 1]], rows1, sem1).wait()
            pltpu.sync_copy(rows1, acc_sh.at[dC.at[k + 1]], add=True)

    def body(i, carry):
        _group(i, srcA, dstA, srcB, dstB, semB, None)

        @pl.when(i < NGRP // 2 - 1)
        def _():
            g = (i + 1) * 2
            pltpu.async_copy(src_hbm.at[w, pl.ds(g * GRP, GRP)], srcA, semA)
            pltpu.async_copy(dst_hbm.at[w, pl.ds(g * GRP, GRP)], dstA, semA)

        _group(i, srcB, dstB, srcA, dstA, semA, NGRP // 2 - 1)

        @pl.when(i < NGRP // 2 - 1)
        def _():
            g = (i + 1) * 2 + 1
            pltpu.async_copy(src_hbm.at[w, pl.ds(g * GRP, GRP)], srcB, semB)
            pltpu.async_copy(dst_hbm.at[w, pl.ds(g * GRP, GRP)], dstB, semB)

        return carry

    lax.fori_loop(0, NGRP // 2, body, 0)
    plsc.subcore_barrier()

    _copy_out_rows(acc_sh, out_hbm, c, s)


def _prescale_body(degp_ref, x_ref, dis_ref, y_ref):
    deg = degp_ref[0, :, 0:1] + degp_ref[1, :, 0:1] + 1.0
    dis = lax.rsqrt(deg)
    dis_ref[...] = dis
    y_ref[...] = x_ref[...] * dis


def _layer1_body(parts_ref, y_ref, dis_ref, w_ref, b_ref, y2_ref):
    dis = dis_ref[...]
    z = (parts_ref[0] + parts_ref[1] + y_ref[...]) * dis
    h = jnp.dot(z, w_ref[...], preferred_element_type=jnp.float32) + b_ref[...]
    y2_ref[...] = jnp.maximum(h, 0.0) * dis


def _layer2_body(parts_ref, y_ref, dis_ref, w_ref, b_ref, wl_ref, bl_ref,
                 out_ref):
    dis = dis_ref[...]
    z = (parts_ref[0] + parts_ref[1] + y_ref[...]) * dis
    h = jnp.dot(z, w_ref[...], preferred_element_type=jnp.float32) + b_ref[...]
    h = jnp.maximum(h, 0.0)
    out_ref[...] = (jnp.dot(h, wl_ref[...], preferred_element_type=jnp.float32)
                    + bl_ref[...])


_BM = 1000
_GRID = N // _BM


def _tc_prescale(degp, xpad):
    return pl.pallas_call(
        _prescale_body,
        grid=(_GRID,),
        in_specs=[
            pl.BlockSpec((NC, _BM, CIN), lambda i: (0, i, 0)),
            pl.BlockSpec((_BM, CIN), lambda i: (i, 0)),
        ],
        out_specs=[
            pl.BlockSpec((_BM, 1), lambda i: (i, 0)),
            pl.BlockSpec((_BM, CIN), lambda i: (i, 0)),
        ],
        out_shape=[
            jax.ShapeDtypeStruct((N, 1), jnp.float32),
            jax.ShapeDtypeStruct((N, CIN), jnp.float32),
        ],
    )(degp, xpad)


def _tc_layer1(parts, y, dis, W, b):
    return pl.pallas_call(
        _layer1_body,
        grid=(_GRID,),
        in_specs=[
            pl.BlockSpec((NC, _BM, CIN), lambda i: (0, i, 0)),
            pl.BlockSpec((_BM, CIN), lambda i: (i, 0)),
            pl.BlockSpec((_BM, 1), lambda i: (i, 0)),
            pl.BlockSpec((CIN, CIN), lambda i: (0, 0)),
            pl.BlockSpec((1, CIN), lambda i: (0, 0)),
        ],
        out_specs=pl.BlockSpec((_BM, CIN), lambda i: (i, 0)),
        out_shape=jax.ShapeDtypeStruct((N, CIN), jnp.float32),
    )(parts, y, dis, W, b)


def _tc_layer2(parts, y, dis, W, b, Wl, bl):
    return pl.pallas_call(
        _layer2_body,
        grid=(_GRID,),
        in_specs=[
            pl.BlockSpec((NC, _BM, CIN), lambda i: (0, i, 0)),
            pl.BlockSpec((_BM, CIN), lambda i: (i, 0)),
            pl.BlockSpec((_BM, 1), lambda i: (i, 0)),
            pl.BlockSpec((CIN, CIN), lambda i: (0, 0)),
            pl.BlockSpec((1, CIN), lambda i: (0, 0)),
            pl.BlockSpec((CIN, CIN), lambda i: (0, 0)),
            pl.BlockSpec((1, CIN), lambda i: (0, 0)),
        ],
        out_specs=pl.BlockSpec((_BM, CIN), lambda i: (i, 0)),
        out_shape=jax.ShapeDtypeStruct((N, CIN), jnp.float32),
    )(parts, y, dis, W, b, Wl, bl)


def kernel(x, edge_index, W1, b1, W2, b2, Wl, bl):
    ei = edge_index.astype(jnp.int32)
    npad = EPAD - E
    pad_src = jnp.arange(npad, dtype=jnp.int32) % N
    pad_dst = N + (jnp.arange(npad, dtype=jnp.int32) % (NPAD - N))
    src3 = jnp.concatenate([ei[0], pad_src]).reshape(NW, CH, CHUNK)
    dst3 = jnp.concatenate([ei[1], pad_dst]).reshape(NW, CH, CHUNK)

    degp = _deg_kernel(dst3)
    dis, y1 = _tc_prescale(degp, x)
    p1 = _spmm_kernel(y1, src3, dst3)
    y2 = _tc_layer1(p1, y1, dis, W1, b1.reshape(1, CIN))
    p2 = _spmm_kernel(y2, src3, dst3)
    return _tc_layer2(p2, y2, dis, W2, b2.reshape(1, CIN), Wl,
                      bl.reshape(1, CIN))


# speedup vs baseline: 1.2959x; 1.0006x over previous
import functools

import jax
import jax.numpy as jnp
from jax import lax
from jax.experimental import pallas as pl
from jax.experimental.pallas import tpu as pltpu
from jax.experimental.pallas import tpu_sc as plsc

N = 10000
CIN = 128
NPAD = 10240
E = 320000
NC = 2
NS = 16
NW = NC * NS
CHUNK = 128
CH = 80
EPT = CH * CHUNK
EPAD = NW * EPT
ROWS_PER_TILE = NPAD // NS
OUT_ROWS_PER_TILE = N // NS

_mesh = plsc.VectorSubcoreMesh(
    core_axis_name="c", subcore_axis_name="s", num_cores=NC, num_subcores=NS)


def _fill_rows(ref, nrows, ncolblk, value, dtype=jnp.float32):
    if dtype == jnp.int16:
        v = jnp.full((2, 16), value, dtype=dtype)

        def row(i, carry):
            for cb in range(ncolblk * 2):
                ref[pl.ds(i * 2, 2), pl.ds(cb * 16, 16)] = v
            return carry

        lax.fori_loop(0, nrows // 2, row, 0)
    else:
        v = jnp.full((16,), value, dtype=dtype)

        def row(i, carry):
            for cb in range(ncolblk):
                ref[i, pl.ds(cb * 16, 16)] = v
            return carry

        lax.fori_loop(0, nrows, row, 0)


def _copy_out_rows(acc_sh, out_hbm, c, s):
    base = pl.multiple_of(s * 624, 8)

    @pl.when(s < NS - 1)
    def _():
        pltpu.sync_copy(acc_sh.at[pl.ds(base, 624)],
                        out_hbm.at[c, pl.ds(base, 624)])

    @pl.when(s == NS - 1)
    def _():
        pltpu.sync_copy(acc_sh.at[pl.ds(15 * 624, N - 15 * 624)],
                        out_hbm.at[c, pl.ds(15 * 624, N - 15 * 624)])


@functools.partial(
    pl.kernel,
    mesh=_mesh,
    out_type=jax.ShapeDtypeStruct((NC, N, CIN), jnp.float32),
    scratch_types=[
        pltpu.VMEM((CH, CHUNK), jnp.int32),
        pltpu.VMEM((CHUNK, CIN), jnp.float32),
        pltpu.VMEM((CHUNK, CIN), jnp.float32),
        pltpu.VMEM_SHARED((NPAD, CIN), jnp.float32),
        pltpu.SemaphoreType.DMA,
    ],
)
def _deg_kernel(dst_hbm, deg_hbm, dstv, ones_v, zeros_v, acc_sh, semz):
    c = lax.axis_index("c")
    s = lax.axis_index("s")
    w = c * NS + s

    _fill_rows(ones_v, CHUNK, CIN // 16, 1.0)
    _fill_rows(zeros_v, CHUNK, CIN // 16, 0.0)
    for k in range(ROWS_PER_TILE // CHUNK):
        pltpu.async_copy(
            zeros_v, acc_sh.at[pl.ds(s * ROWS_PER_TILE + k * CHUNK, CHUNK)],
            semz)
    pltpu.sync_copy(dst_hbm.at[w], dstv)
    for k in range(ROWS_PER_TILE // CHUNK):
        pltpu.make_async_copy(
            zeros_v, acc_sh.at[pl.ds(s * ROWS_PER_TILE + k * CHUNK, CHUNK)],
            semz).wait()
    plsc.subcore_barrier()

    def body(j, carry):
        pltpu.sync_copy(ones_v, acc_sh.at[dstv.at[j]], add=True)
        return carry

    lax.fori_loop(0, CH, body, 0)
    plsc.subcore_barrier()

    _copy_out_rows(acc_sh, deg_hbm, c, s)


GRP = 8
NGRP = CH // GRP


@functools.partial(
    pl.kernel,
    mesh=_mesh,
    out_type=jax.ShapeDtypeStruct((NC, N, CIN), jnp.float32),
    scratch_types=[
        pltpu.VMEM((GRP, CHUNK), jnp.int32),
        pltpu.VMEM((GRP, CHUNK), jnp.int32),
        pltpu.VMEM((GRP, CHUNK), jnp.int32),
        pltpu.VMEM((GRP, CHUNK), jnp.int32),
        pltpu.VMEM((CHUNK, CIN), jnp.float32),
        pltpu.VMEM((CHUNK, CIN), jnp.float32),
        pltpu.VMEM_SHARED((NPAD, CIN), jnp.float32),
        pltpu.SemaphoreType.DMA,
        pltpu.SemaphoreType.DMA,
        pltpu.SemaphoreType.DMA,
        pltpu.SemaphoreType.DMA,
    ],
)
def _spmm_kernel(y_hbm, src_hbm, dst_hbm, out_hbm, srcA, dstA, srcB, dstB,
                 rows0, rows1, acc_sh, sem0, sem1, semA, semB):
    c = lax.axis_index("c")
    s = lax.axis_index("s")
    w = c * NS + s

    _fill_rows(rows0, CHUNK, CIN // 16, 0.0)
    for k in range(ROWS_PER_TILE // CHUNK):
        pltpu.async_copy(
            rows0, acc_sh.at[pl.ds(s * ROWS_PER_TILE + k * CHUNK, CHUNK)],
            semA)
    pltpu.sync_copy(src_hbm.at[w, pl.ds(0, GRP)], srcA)
    pltpu.sync_copy(dst_hbm.at[w, pl.ds(0, GRP)], dstA)
    pltpu.async_copy(src_hbm.at[w, pl.ds(GRP, GRP)], srcB, semB)
    pltpu.async_copy(dst_hbm.at[w, pl.ds(GRP, GRP)], dstB, semB)
    for k in range(ROWS_PER_TILE // CHUNK):
        pltpu.make_async_copy(
            rows0, acc_sh.at[pl.ds(s * ROWS_PER_TILE + k * CHUNK, CHUNK)],
            semA).wait()
    pltpu.async_copy(y_hbm.at[srcA.at[0, pl.ds(0, 64)]],
                     rows0.at[pl.ds(0, 64)], sem0)
    pltpu.async_copy(y_hbm.at[srcA.at[0, pl.ds(64, 64)]],
                     rows0.at[pl.ds(64, 64)], sem0)
    plsc.subcore_barrier()

    def _wait_idx(sC, dC, g, sem):
        pltpu.make_async_copy(src_hbm.at[w, pl.ds(g * GRP, GRP)], sC, sem).wait()
        pltpu.make_async_copy(dst_hbm.at[w, pl.ds(g * GRP, GRP)], dC, sem).wait()

    def _group(i, sC, dC, sN, dN, semN, last):
        def _gather(sX, j, rows, sem):
            pltpu.async_copy(y_hbm.at[sX.at[j, pl.ds(0, 64)]],
                             rows.at[pl.ds(0, 64)], sem)
            pltpu.async_copy(y_hbm.at[sX.at[j, pl.ds(64, 64)]],
                             rows.at[pl.ds(64, 64)], sem)

        def _gwait(sX, j, rows, sem):
            pltpu.make_async_copy(y_hbm.at[sX.at[j, pl.ds(0, 64)]],
                                  rows.at[pl.ds(0, 64)], sem).wait()
            pltpu.make_async_copy(y_hbm.at[sX.at[j, pl.ds(64, 64)]],
                                  rows.at[pl.ds(64, 64)], sem).wait()

        for k in range(0, GRP, 2):
            _gather(sC, k + 1, rows1, sem1)
            _gwait(sC, k, rows0, sem0)
            pltpu.sync_copy(rows0, acc_sh.at[dC.at[k]], add=True)
            if k + 2 < GRP:
                _gather(sC, k + 2, rows0, sem0)
            elif last is None:
                _wait_idx(sN, dN, 0, semN)
                _gather(sN, 0, rows0, sem0)
            else:

                @pl.when(i < last)
                def _():
                    _wait_idx(sN, dN, 0, semN)
                    _gather(sN, 0, rows0, sem0)

            _gwait(sC, k + 1, rows1, sem1)
            pltpu.sync_copy(rows1, acc_sh.at[dC.at[k + 1]], add=True)

    def body(i, carry):
        _group(i, srcA, dstA, srcB, dstB, semB, None)

        @pl.when(i < NGRP // 2 - 1)
        def _():
            g = (i + 1) * 2
            pltpu.async_copy(src_hbm.at[w, pl.ds(g * GRP, GRP)], srcA, semA)
            pltpu.async_copy(dst_hbm.at[w, pl.ds(g * GRP, GRP)], dstA, semA)

        _group(i, srcB, dstB, srcA, dstA, semA, NGRP // 2 - 1)

        @pl.when(i < NGRP // 2 - 1)
        def _():
            g = (i + 1) * 2 + 1
            pltpu.async_copy(src_hbm.at[w, pl.ds(g * GRP, GRP)], srcB, semB)
            pltpu.async_copy(dst_hbm.at[w, pl.ds(g * GRP, GRP)], dstB, semB)

        return carry

    lax.fori_loop(0, NGRP // 2, body, 0)
    plsc.subcore_barrier()

    _copy_out_rows(acc_sh, out_hbm, c, s)


def _prescale_body(degp_ref, x_ref, dis_ref, y_ref):
    deg = degp_ref[0, :, 0:1] + degp_ref[1, :, 0:1] + 1.0
    dis = lax.rsqrt(deg)
    dis_ref[...] = dis
    y_ref[...] = x_ref[...] * dis


def _layer1_body(parts_ref, y_ref, dis_ref, w_ref, b_ref, y2_ref):
    dis = dis_ref[...]
    z = (parts_ref[0] + parts_ref[1] + y_ref[...]) * dis
    h = jnp.dot(z, w_ref[...], preferred_element_type=jnp.float32) + b_ref[...]
    y2_ref[...] = jnp.maximum(h, 0.0) * dis


def _layer2_body(parts_ref, y_ref, dis_ref, w_ref, b_ref, wl_ref, bl_ref,
                 out_ref):
    dis = dis_ref[...]
    z = (parts_ref[0] + parts_ref[1] + y_ref[...]) * dis
    h = jnp.dot(z, w_ref[...], preferred_element_type=jnp.float32) + b_ref[...]
    h = jnp.maximum(h, 0.0)
    out_ref[...] = (jnp.dot(h, wl_ref[...], preferred_element_type=jnp.float32)
                    + bl_ref[...])


_BM = 1000
_GRID = N // _BM


def _tc_prescale(degp, xpad):
    return pl.pallas_call(
        _prescale_body,
        grid=(_GRID,),
        in_specs=[
            pl.BlockSpec((NC, _BM, CIN), lambda i: (0, i, 0)),
            pl.BlockSpec((_BM, CIN), lambda i: (i, 0)),
        ],
        out_specs=[
            pl.BlockSpec((_BM, 1), lambda i: (i, 0)),
            pl.BlockSpec((_BM, CIN), lambda i: (i, 0)),
        ],
        out_shape=[
            jax.ShapeDtypeStruct((N, 1), jnp.float32),
            jax.ShapeDtypeStruct((N, CIN), jnp.float32),
        ],
    )(degp, xpad)


def _tc_layer1(parts, y, dis, W, b):
    return pl.pallas_call(
        _layer1_body,
        grid=(_GRID,),
        in_specs=[
            pl.BlockSpec((NC, _BM, CIN), lambda i: (0, i, 0)),
            pl.BlockSpec((_BM, CIN), lambda i: (i, 0)),
            pl.BlockSpec((_BM, 1), lambda i: (i, 0)),
            pl.BlockSpec((CIN, CIN), lambda i: (0, 0)),
            pl.BlockSpec((1, CIN), lambda i: (0, 0)),
        ],
        out_specs=pl.BlockSpec((_BM, CIN), lambda i: (i, 0)),
        out_shape=jax.ShapeDtypeStruct((N, CIN), jnp.float32),
    )(parts, y, dis, W, b)


def _tc_layer2(parts, y, dis, W, b, Wl, bl):
    return pl.pallas_call(
        _layer2_body,
        grid=(_GRID,),
        in_specs=[
            pl.BlockSpec((NC, _BM, CIN), lambda i: (0, i, 0)),
            pl.BlockSpec((_BM, CIN), lambda i: (i, 0)),
            pl.BlockSpec((_BM, 1), lambda i: (i, 0)),
            pl.BlockSpec((CIN, CIN), lambda i: (0, 0)),
            pl.BlockSpec((1, CIN), lambda i: (0, 0)),
            pl.BlockSpec((CIN, CIN), lambda i: (0, 0)),
            pl.BlockSpec((1, CIN), lambda i: (0, 0)),
        ],
        out_specs=pl.BlockSpec((_BM, CIN), lambda i: (i, 0)),
        out_shape=jax.ShapeDtypeStruct((N, CIN), jnp.float32),
    )(parts, y, dis, W, b, Wl, bl)


def kernel(x, edge_index, W1, b1, W2, b2, Wl, bl):
    ei = edge_index.astype(jnp.int32)
    npad = EPAD - E
    pad_src = jnp.arange(npad, dtype=jnp.int32) % N
    pad_dst = N + (jnp.arange(npad, dtype=jnp.int32) % (NPAD - N))
    src3 = jnp.concatenate([ei[0], pad_src]).reshape(NW, CH, CHUNK)
    dst3 = jnp.concatenate([ei[1], pad_dst]).reshape(NW, CH, CHUNK)

    degp = _deg_kernel(dst3)
    dis, y1 = _tc_prescale(degp, x)
    p1 = _spmm_kernel(y1, src3, dst3)
    y2 = _tc_layer1(p1, y1, dis, W1, b1.reshape(1, CIN))
    p2 = _spmm_kernel(y2, src3, dst3)
    return _tc_layer2(p2, y2, dis, W2, b2.reshape(1, CIN), Wl,
                      bl.reshape(1, CIN))


# speedup vs baseline: 1.2965x; 1.0004x over previous
import functools

import jax
import jax.numpy as jnp
from jax import lax
from jax.experimental import pallas as pl
from jax.experimental.pallas import tpu as pltpu
from jax.experimental.pallas import tpu_sc as plsc

N = 10000
CIN = 128
NPAD = 10240
E = 320000
NC = 2
NS = 16
NW = NC * NS
CHUNK = 128
CH = 80
EPT = CH * CHUNK
EPAD = NW * EPT
ROWS_PER_TILE = NPAD // NS
OUT_ROWS_PER_TILE = N // NS

_mesh = plsc.VectorSubcoreMesh(
    core_axis_name="c", subcore_axis_name="s", num_cores=NC, num_subcores=NS)


def _fill_rows(ref, nrows, ncolblk, value, dtype=jnp.float32):
    if dtype == jnp.int16:
        v = jnp.full((2, 16), value, dtype=dtype)

        def row(i, carry):
            for cb in range(ncolblk * 2):
                ref[pl.ds(i * 2, 2), pl.ds(cb * 16, 16)] = v
            return carry

        lax.fori_loop(0, nrows // 2, row, 0)
    else:
        v = jnp.full((16,), value, dtype=dtype)

        def row(i, carry):
            for cb in range(ncolblk):
                ref[i, pl.ds(cb * 16, 16)] = v
            return carry

        lax.fori_loop(0, nrows, row, 0)


def _copy_out_rows(acc_sh, out_hbm, c, s):
    base = pl.multiple_of(s * 624, 8)

    @pl.when(s < NS - 1)
    def _():
        pltpu.sync_copy(acc_sh.at[pl.ds(base, 624)],
                        out_hbm.at[c, pl.ds(base, 624)])

    @pl.when(s == NS - 1)
    def _():
        pltpu.sync_copy(acc_sh.at[pl.ds(15 * 624, N - 15 * 624)],
                        out_hbm.at[c, pl.ds(15 * 624, N - 15 * 624)])


@functools.partial(
    pl.kernel,
    mesh=_mesh,
    out_type=jax.ShapeDtypeStruct((NC, N, CIN), jnp.float32),
    scratch_types=[
        pltpu.VMEM((CH, CHUNK), jnp.int32),
        pltpu.VMEM((CHUNK, CIN), jnp.float32),
        pltpu.VMEM((CHUNK, CIN), jnp.float32),
        pltpu.VMEM_SHARED((NPAD, CIN), jnp.float32),
        pltpu.SemaphoreType.DMA,
    ],
)
def _deg_kernel(dst_hbm, deg_hbm, dstv, ones_v, zeros_v, acc_sh, semz):
    c = lax.axis_index("c")
    s = lax.axis_index("s")
    w = c * NS + s

    _fill_rows(ones_v, CHUNK, CIN // 16, 1.0)
    _fill_rows(zeros_v, CHUNK, CIN // 16, 0.0)
    for k in range(ROWS_PER_TILE // CHUNK):
        pltpu.async_copy(
            zeros_v, acc_sh.at[pl.ds(s * ROWS_PER_TILE + k * CHUNK, CHUNK)],
            semz)
    pltpu.sync_copy(dst_hbm.at[w], dstv)
    for k in range(ROWS_PER_TILE // CHUNK):
        pltpu.make_async_copy(
            zeros_v, acc_sh.at[pl.ds(s * ROWS_PER_TILE + k * CHUNK, CHUNK)],
            semz).wait()
    plsc.subcore_barrier()

    def body(j, carry):
        pltpu.async_copy(ones_v, acc_sh.at[dstv.at[j]], semz, add=True)
        return carry

    lax.fori_loop(0, CH, body, 0)

    def drain(j, carry):
        pltpu.make_async_copy(ones_v, acc_sh.at[dstv.at[j]], semz).wait()
        return carry

    lax.fori_loop(0, CH, drain, 0)
    plsc.subcore_barrier()

    _copy_out_rows(acc_sh, deg_hbm, c, s)


GRP = 8
NGRP = CH // GRP


@functools.partial(
    pl.kernel,
    mesh=_mesh,
    out_type=jax.ShapeDtypeStruct((NC, N, CIN), jnp.float32),
    scratch_types=[
        pltpu.VMEM((GRP, CHUNK), jnp.int32),
        pltpu.VMEM((GRP, CHUNK), jnp.int32),
        pltpu.VMEM((GRP, CHUNK), jnp.int32),
        pltpu.VMEM((GRP, CHUNK), jnp.int32),
        pltpu.VMEM((CHUNK, CIN), jnp.float32),
        pltpu.VMEM((CHUNK, CIN), jnp.float32),
        pltpu.VMEM_SHARED((NPAD, CIN), jnp.float32),
        pltpu.SemaphoreType.DMA,
        pltpu.SemaphoreType.DMA,
        pltpu.SemaphoreType.DMA,
        pltpu.SemaphoreType.DMA,
    ],
)
def _spmm_kernel(y_hbm, src_hbm, dst_hbm, out_hbm, srcA, dstA, srcB, dstB,
                 rows0, rows1, acc_sh, sem0, sem1, semA, semB):
    c = lax.axis_index("c")
    s = lax.axis_index("s")
    w = c * NS + s

    _fill_rows(rows0, CHUNK, CIN // 16, 0.0)
    for k in range(ROWS_PER_TILE // CHUNK):
        pltpu.async_copy(
            rows0, acc_sh.at[pl.ds(s * ROWS_PER_TILE + k * CHUNK, CHUNK)],
            semA)
    pltpu.sync_copy(src_hbm.at[w, pl.ds(0, GRP)], srcA)
    pltpu.sync_copy(dst_hbm.at[w, pl.ds(0, GRP)], dstA)
    pltpu.async_copy(src_hbm.at[w, pl.ds(GRP, GRP)], srcB, semB)
    pltpu.async_copy(dst_hbm.at[w, pl.ds(GRP, GRP)], dstB, semB)
    for k in range(ROWS_PER_TILE // CHUNK):
        pltpu.make_async_copy(
            rows0, acc_sh.at[pl.ds(s * ROWS_PER_TILE + k * CHUNK, CHUNK)],
            semA).wait()
    pltpu.async_copy(y_hbm.at[srcA.at[0]], rows0, sem0)
    plsc.subcore_barrier()

    def _wait_idx(sC, dC, g, sem):
        pltpu.make_async_copy(src_hbm.at[w, pl.ds(g * GRP, GRP)], sC, sem).wait()
        pltpu.make_async_copy(dst_hbm.at[w, pl.ds(g * GRP, GRP)], dC, sem).wait()

    def _group(i, sC, dC, sN, dN, semN, last):
        for k in range(0, GRP, 2):
            pltpu.async_copy(y_hbm.at[sC.at[k + 1]], rows1, sem1)
            pltpu.make_async_copy(y_hbm.at[sC.at[k]], rows0, sem0).wait()
            pltpu.sync_copy(rows0, acc_sh.at[dC.at[k]], add=True)
            if k + 2 < GRP:
                pltpu.async_copy(y_hbm.at[sC.at[k + 2]], rows0, sem0)
            elif last is None:
                _wait_idx(sN, dN, 0, semN)
                pltpu.async_copy(y_hbm.at[sN.at[0]], rows0, sem0)
            else:

                @pl.when(i < last)
                def _():
                    _wait_idx(sN, dN, 0, semN)
                    pltpu.async_copy(y_hbm.at[sN.at[0]], rows0, sem0)

            pltpu.make_async_copy(y_hbm.at[sC.at[k + 1]], rows1, sem1).wait()
            pltpu.sync_copy(rows1, acc_sh.at[dC.at[k + 1]], add=True)

    def body(i, carry):
        _group(i, srcA, dstA, srcB, dstB, semB, None)

        @pl.when(i < NGRP // 2 - 1)
        def _():
            g = (i + 1) * 2
            pltpu.async_copy(src_hbm.at[w, pl.ds(g * GRP, GRP)], srcA, semA)
            pltpu.async_copy(dst_hbm.at[w, pl.ds(g * GRP, GRP)], dstA, semA)

        _group(i, srcB, dstB, srcA, dstA, semA, NGRP // 2 - 1)

        @pl.when(i < NGRP // 2 - 1)
        def _():
            g = (i + 1) * 2 + 1
            pltpu.async_copy(src_hbm.at[w, pl.ds(g * GRP, GRP)], srcB, semB)
            pltpu.async_copy(dst_hbm.at[w, pl.ds(g * GRP, GRP)], dstB, semB)

        return carry

    lax.fori_loop(0, NGRP // 2, body, 0)
    plsc.subcore_barrier()

    _copy_out_rows(acc_sh, out_hbm, c, s)


def _prescale_body(degp_ref, x_ref, dis_ref, y_ref):
    deg = degp_ref[0, :, 0:1] + degp_ref[1, :, 0:1] + 1.0
    dis = lax.rsqrt(deg)
    dis_ref[...] = dis
    y_ref[...] = x_ref[...] * dis


def _layer1_body(parts_ref, y_ref, dis_ref, w_ref, b_ref, y2_ref):
    dis = dis_ref[...]
    z = (parts_ref[0] + parts_ref[1] + y_ref[...]) * dis
    h = jnp.dot(z, w_ref[...], preferred_element_type=jnp.float32) + b_ref[...]
    y2_ref[...] = jnp.maximum(h, 0.0) * dis


def _layer2_body(parts_ref, y_ref, dis_ref, w_ref, b_ref, wl_ref, bl_ref,
                 out_ref):
    dis = dis_ref[...]
    z = (parts_ref[0] + parts_ref[1] + y_ref[...]) * dis
    h = jnp.dot(z, w_ref[...], preferred_element_type=jnp.float32) + b_ref[...]
    h = jnp.maximum(h, 0.0)
    out_ref[...] = (jnp.dot(h, wl_ref[...], preferred_element_type=jnp.float32)
                    + bl_ref[...])


_BM = 1000
_GRID = N // _BM


def _tc_prescale(degp, xpad):
    return pl.pallas_call(
        _prescale_body,
        grid=(_GRID,),
        in_specs=[
            pl.BlockSpec((NC, _BM, CIN), lambda i: (0, i, 0)),
            pl.BlockSpec((_BM, CIN), lambda i: (i, 0)),
        ],
        out_specs=[
            pl.BlockSpec((_BM, 1), lambda i: (i, 0)),
            pl.BlockSpec((_BM, CIN), lambda i: (i, 0)),
        ],
        out_shape=[
            jax.ShapeDtypeStruct((N, 1), jnp.float32),
            jax.ShapeDtypeStruct((N, CIN), jnp.float32),
        ],
    )(degp, xpad)


def _tc_layer1(parts, y, dis, W, b):
    return pl.pallas_call(
        _layer1_body,
        grid=(_GRID,),
        in_specs=[
            pl.BlockSpec((NC, _BM, CIN), lambda i: (0, i, 0)),
            pl.BlockSpec((_BM, CIN), lambda i: (i, 0)),
            pl.BlockSpec((_BM, 1), lambda i: (i, 0)),
            pl.BlockSpec((CIN, CIN), lambda i: (0, 0)),
            pl.BlockSpec((1, CIN), lambda i: (0, 0)),
        ],
        out_specs=pl.BlockSpec((_BM, CIN), lambda i: (i, 0)),
        out_shape=jax.ShapeDtypeStruct((N, CIN), jnp.float32),
    )(parts, y, dis, W, b)


def _tc_layer2(parts, y, dis, W, b, Wl, bl):
    return pl.pallas_call(
        _layer2_body,
        grid=(_GRID,),
        in_specs=[
            pl.BlockSpec((NC, _BM, CIN), lambda i: (0, i, 0)),
            pl.BlockSpec((_BM, CIN), lambda i: (i, 0)),
            pl.BlockSpec((_BM, 1), lambda i: (i, 0)),
            pl.BlockSpec((CIN, CIN), lambda i: (0, 0)),
            pl.BlockSpec((1, CIN), lambda i: (0, 0)),
            pl.BlockSpec((CIN, CIN), lambda i: (0, 0)),
            pl.BlockSpec((1, CIN), lambda i: (0, 0)),
        ],
        out_specs=pl.BlockSpec((_BM, CIN), lambda i: (i, 0)),
        out_shape=jax.ShapeDtypeStruct((N, CIN), jnp.float32),
    )(parts, y, dis, W, b, Wl, bl)


def kernel(x, edge_index, W1, b1, W2, b2, Wl, bl):
    ei = edge_index.astype(jnp.int32)
    npad = EPAD - E
    pad_src = jnp.arange(npad, dtype=jnp.int32) % N
    pad_dst = N + (jnp.arange(npad, dtype=jnp.int32) % (NPAD - N))
    src3 = jnp.concatenate([ei[0], pad_src]).reshape(NW, CH, CHUNK)
    dst3 = jnp.concatenate([ei[1], pad_dst]).reshape(NW, CH, CHUNK)

    degp = _deg_kernel(dst3)
    dis, y1 = _tc_prescale(degp, x)
    p1 = _spmm_kernel(y1, src3, dst3)
    y2 = _tc_layer1(p1, y1, dis, W1, b1.reshape(1, CIN))
    p2 = _spmm_kernel(y2, src3, dst3)
    return _tc_layer2(p2, y2, dis, W2, b2.reshape(1, CIN), Wl,
                      bl.reshape(1, CIN))


# speedup vs baseline: 1.3006x; 1.0032x over previous
import functools

import jax
import jax.numpy as jnp
from jax import lax
from jax.experimental import pallas as pl
from jax.experimental.pallas import tpu as pltpu
from jax.experimental.pallas import tpu_sc as plsc

N = 10000
CIN = 128
NPAD = 10240
E = 320000
NC = 2
NS = 16
NW = NC * NS
CHUNK = 128
CH = 80
EPT = CH * CHUNK
EPAD = NW * EPT
ROWS_PER_TILE = NPAD // NS

_mesh = plsc.VectorSubcoreMesh(
    core_axis_name="c", subcore_axis_name="s", num_cores=NC, num_subcores=NS)


def _fill_rows(ref, nrows, ncolblk, value):
    v = jnp.full((16,), value, dtype=jnp.float32)

    def row(i, carry):
        for cb in range(ncolblk):
            ref[i, pl.ds(cb * 16, 16)] = v
        return carry

    lax.fori_loop(0, nrows, row, 0)


def _copy_out_rows(acc_sh, out_hbm, c, s):
    base = pl.multiple_of(s * 624, 8)

    @pl.when(s < NS - 1)
    def _():
        pltpu.sync_copy(acc_sh.at[pl.ds(base, 624)],
                        out_hbm.at[c, pl.ds(base, 624)])

    @pl.when(s == NS - 1)
    def _():
        pltpu.sync_copy(acc_sh.at[pl.ds(15 * 624, N - 15 * 624)],
                        out_hbm.at[c, pl.ds(15 * 624, N - 15 * 624)])


@functools.partial(
    pl.kernel,
    mesh=_mesh,
    out_type=jax.ShapeDtypeStruct((NC, N, CIN), jnp.float32),
    scratch_types=[
        pltpu.VMEM((CH, CHUNK), jnp.int32),
        pltpu.VMEM((CHUNK, CIN), jnp.float32),
        pltpu.VMEM((CHUNK, CIN), jnp.float32),
        pltpu.VMEM_SHARED((NPAD, CIN), jnp.float32),
        pltpu.SemaphoreType.DMA,
    ],
)
def _deg_kernel(dst_hbm, deg_hbm, dstv, ones_v, zeros_v, acc_sh, semz):
    c = lax.axis_index("c")
    s = lax.axis_index("s")
    w = c * NS + s

    _fill_rows(ones_v, CHUNK, CIN // 16, 1.0)
    _fill_rows(zeros_v, CHUNK, CIN // 16, 0.0)
    for k in range(ROWS_PER_TILE // CHUNK):
        pltpu.async_copy(
            zeros_v, acc_sh.at[pl.ds(s * ROWS_PER_TILE + k * CHUNK, CHUNK)],
            semz)
    pltpu.sync_copy(dst_hbm.at[w], dstv)
    for k in range(ROWS_PER_TILE // CHUNK):
        pltpu.make_async_copy(
            zeros_v, acc_sh.at[pl.ds(s * ROWS_PER_TILE + k * CHUNK, CHUNK)],
            semz).wait()
    plsc.subcore_barrier()

    def body(j, carry):
        pltpu.async_copy(ones_v, acc_sh.at[dstv.at[j]], semz, add=True)
        return carry

    lax.fori_loop(0, CH, body, 0)

    def drain(j, carry):
        pltpu.make_async_copy(ones_v, acc_sh.at[dstv.at[j]], semz).wait()
        return carry

    lax.fori_loop(0, CH, drain, 0)
    plsc.subcore_barrier()

    _copy_out_rows(acc_sh, deg_hbm, c, s)


GRP = 8
NGRP = CH // GRP


@functools.partial(
    pl.kernel,
    mesh=_mesh,
    out_type=jax.ShapeDtypeStruct((NC, N, CIN), jnp.float32),
    scratch_types=[
        pltpu.VMEM((GRP, CHUNK), jnp.int32),
        pltpu.VMEM((GRP, CHUNK), jnp.int32),
        pltpu.VMEM((GRP, CHUNK), jnp.int32),
        pltpu.VMEM((GRP, CHUNK), jnp.int32),
        pltpu.VMEM((CHUNK, CIN), jnp.float32),
        pltpu.VMEM((CHUNK, CIN), jnp.float32),
        pltpu.VMEM_SHARED((NPAD, CIN), jnp.float32),
        pltpu.SemaphoreType.DMA,
        pltpu.SemaphoreType.DMA,
        pltpu.SemaphoreType.DMA,
        pltpu.SemaphoreType.DMA,
    ],
)
def _spmm_kernel(y_hbm, src_hbm, dst_hbm, out_hbm, srcA, dstA, srcB, dstB,
                 rows0, rows1, acc_sh, sem0, sem1, semA, semB):
    c = lax.axis_index("c")
    s = lax.axis_index("s")
    w = c * NS + s

    _fill_rows(rows0, CHUNK, CIN // 16, 0.0)
    for k in range(ROWS_PER_TILE // CHUNK):
        pltpu.async_copy(
            rows0, acc_sh.at[pl.ds(s * ROWS_PER_TILE + k * CHUNK, CHUNK)],
            semA)
    pltpu.sync_copy(src_hbm.at[w, pl.ds(0, GRP)], srcA)
    pltpu.sync_copy(dst_hbm.at[w, pl.ds(0, GRP)], dstA)
    pltpu.async_copy(src_hbm.at[w, pl.ds(GRP, GRP)], srcB, semB)
    pltpu.async_copy(dst_hbm.at[w, pl.ds(GRP, GRP)], dstB, semB)
    for k in range(ROWS_PER_TILE // CHUNK):
        pltpu.make_async_copy(
            rows0, acc_sh.at[pl.ds(s * ROWS_PER_TILE + k * CHUNK, CHUNK)],
            semA).wait()
    pltpu.async_copy(y_hbm.at[srcA.at[0]], rows0, sem0)
    plsc.subcore_barrier()

    def _wait_idx(sC, dC, g, sem):
        pltpu.make_async_copy(src_hbm.at[w, pl.ds(g * GRP, GRP)], sC, sem).wait()
        pltpu.make_async_copy(dst_hbm.at[w, pl.ds(g * GRP, GRP)], dC, sem).wait()

    def _group(i, sC, dC, sN, dN, semN, last):
        for k in range(0, GRP, 2):
            pltpu.async_copy(y_hbm.at[sC.at[k + 1]], rows1, sem1)
            pltpu.make_async_copy(y_hbm.at[sC.at[k]], rows0, sem0).wait()
            pltpu.sync_copy(rows0, acc_sh.at[dC.at[k]], add=True)
            if k + 2 < GRP:
                pltpu.async_copy(y_hbm.at[sC.at[k + 2]], rows0, sem0)
            elif last is None:
                _wait_idx(sN, dN, 0, semN)
                pltpu.async_copy(y_hbm.at[sN.at[0]], rows0, sem0)
            else:

                @pl.when(i < last)
                def _():
                    _wait_idx(sN, dN, 0, semN)
                    pltpu.async_copy(y_hbm.at[sN.at[0]], rows0, sem0)

            pltpu.make_async_copy(y_hbm.at[sC.at[k + 1]], rows1, sem1).wait()
            pltpu.sync_copy(rows1, acc_sh.at[dC.at[k + 1]], add=True)

    def body(i, carry):
        _group(i, srcA, dstA, srcB, dstB, semB, None)

        @pl.when(i < NGRP // 2 - 1)
        def _():
            g = (i + 1) * 2
            pltpu.async_copy(src_hbm.at[w, pl.ds(g * GRP, GRP)], srcA, semA)
            pltpu.async_copy(dst_hbm.at[w, pl.ds(g * GRP, GRP)], dstA, semA)

        _group(i, srcB, dstB, srcA, dstA, semA, NGRP // 2 - 1)

        @pl.when(i < NGRP // 2 - 1)
        def _():
            g = (i + 1) * 2 + 1
            pltpu.async_copy(src_hbm.at[w, pl.ds(g * GRP, GRP)], srcB, semB)
            pltpu.async_copy(dst_hbm.at[w, pl.ds(g * GRP, GRP)], dstB, semB)

        return carry

    lax.fori_loop(0, NGRP // 2, body, 0)
    plsc.subcore_barrier()

    _copy_out_rows(acc_sh, out_hbm, c, s)


def _prescale_body(degp_ref, x_ref, dis_ref, y_ref):
    deg = degp_ref[0, :, 0:1] + degp_ref[1, :, 0:1] + 1.0
    dis = lax.rsqrt(deg)
    dis_ref[...] = dis
    y_ref[...] = x_ref[...] * dis


def _layer1_body(parts_ref, y_ref, dis_ref, w_ref, b_ref, y2_ref):
    dis = dis_ref[...]
    z = (parts_ref[0] + parts_ref[1] + y_ref[...]) * dis
    h = jnp.dot(z, w_ref[...], preferred_element_type=jnp.float32) + b_ref[...]
    y2_ref[...] = jnp.maximum(h, 0.0) * dis


def _layer2_body(parts_ref, y_ref, dis_ref, w_ref, b_ref, wl_ref, bl_ref,
                 out_ref):
    dis = dis_ref[...]
    z = (parts_ref[0] + parts_ref[1] + y_ref[...]) * dis
    h = jnp.dot(z, w_ref[...], preferred_element_type=jnp.float32) + b_ref[...]
    h = jnp.maximum(h, 0.0)
    out_ref[...] = (jnp.dot(h, wl_ref[...], preferred_element_type=jnp.float32)
                    + bl_ref[...])


_BM = 1000
_GRID = N // _BM


def _tc_prescale(degp, xpad):
    return pl.pallas_call(
        _prescale_body,
        grid=(_GRID,),
        in_specs=[
            pl.BlockSpec((NC, _BM, CIN), lambda i: (0, i, 0)),
            pl.BlockSpec((_BM, CIN), lambda i: (i, 0)),
        ],
        out_specs=[
            pl.BlockSpec((_BM, 1), lambda i: (i, 0)),
            pl.BlockSpec((_BM, CIN), lambda i: (i, 0)),
        ],
        out_shape=[
            jax.ShapeDtypeStruct((N, 1), jnp.float32),
            jax.ShapeDtypeStruct((N, CIN), jnp.float32),
        ],
    )(degp, xpad)


def _tc_layer1(parts, y, dis, W, b):
    return pl.pallas_call(
        _layer1_body,
        grid=(_GRID,),
        in_specs=[
            pl.BlockSpec((NC, _BM, CIN), lambda i: (0, i, 0)),
            pl.BlockSpec((_BM, CIN), lambda i: (i, 0)),
            pl.BlockSpec((_BM, 1), lambda i: (i, 0)),
            pl.BlockSpec((CIN, CIN), lambda i: (0, 0)),
            pl.BlockSpec((1, CIN), lambda i: (0, 0)),
        ],
        out_specs=pl.BlockSpec((_BM, CIN), lambda i: (i, 0)),
        out_shape=jax.ShapeDtypeStruct((N, CIN), jnp.float32),
    )(parts, y, dis, W, b)


def _tc_layer2(parts, y, dis, W, b, Wl, bl):
    return pl.pallas_call(
        _layer2_body,
        grid=(_GRID,),
        in_specs=[
            pl.BlockSpec((NC, _BM, CIN), lambda i: (0, i, 0)),
            pl.BlockSpec((_BM, CIN), lambda i: (i, 0)),
            pl.BlockSpec((_BM, 1), lambda i: (i, 0)),
            pl.BlockSpec((CIN, CIN), lambda i: (0, 0)),
            pl.BlockSpec((1, CIN), lambda i: (0, 0)),
            pl.BlockSpec((CIN, CIN), lambda i: (0, 0)),
            pl.BlockSpec((1, CIN), lambda i: (0, 0)),
        ],
        out_specs=pl.BlockSpec((_BM, CIN), lambda i: (i, 0)),
        out_shape=jax.ShapeDtypeStruct((N, CIN), jnp.float32),
    )(parts, y, dis, W, b, Wl, bl)


def kernel(x, edge_index, W1, b1, W2, b2, Wl, bl):
    ei = edge_index.astype(jnp.int32)
    npad = EPAD - E
    pad_src = jnp.arange(npad, dtype=jnp.int32) % N
    pad_dst = N + (jnp.arange(npad, dtype=jnp.int32) % (NPAD - N))
    src3 = jnp.concatenate([ei[0], pad_src]).reshape(NW, CH, CHUNK)
    dst3 = jnp.concatenate([ei[1], pad_dst]).reshape(NW, CH, CHUNK)

    degp = _deg_kernel(dst3)
    dis, y1 = _tc_prescale(degp, x)
    p1 = _spmm_kernel(y1, src3, dst3)
    y2 = _tc_layer1(p1, y1, dis, W1, b1.reshape(1, CIN))
    p2 = _spmm_kernel(y2, src3, dst3)
    return _tc_layer2(p2, y2, dis, W2, b2.reshape(1, CIN), Wl,
                      bl.reshape(1, CIN))
